# Initial kernel scaffold; baseline (speedup 1.0000x reference)
import jax
import jax.numpy as jnp
from jax.experimental import pallas as pl


def kernel(outputs, targets):
    raise NotImplementedError("write your pallas kernel here")


# speedup vs baseline: 3.1724x; 3.1724x over previous
import jax
import jax.numpy as jnp
from jax.experimental import pallas as pl
from jax.experimental.pallas import tpu as pltpu

ROW_BLK = 64


def _ce_rows_kernel(x_ref, t_ref, out_ref):
    x = x_ref[...]
    t = t_ref[...]
    col = jax.lax.broadcasted_iota(jnp.int32, x.shape, 1)
    picked = jnp.sum(jnp.where(col == t, x, 0.0), axis=1, keepdims=True)
    m = jnp.max(x, axis=1, keepdims=True)
    s = jnp.sum(jnp.exp(x - m), axis=1, keepdims=True)
    out_ref[...] = picked - m - jnp.log(s)


def _mean_kernel(x_ref, out_ref):
    out_ref[0, 0] = -jnp.mean(x_ref[...])


def kernel(outputs, targets):
    B, V = outputs.shape
    t2 = targets.astype(jnp.int32).reshape(B, 1)
    logp = pl.pallas_call(
        _ce_rows_kernel,
        grid=(B // ROW_BLK,),
        in_specs=[
            pl.BlockSpec((ROW_BLK, V), lambda i: (i, 0)),
            pl.BlockSpec((ROW_BLK, 1), lambda i: (i, 0)),
        ],
        out_specs=pl.BlockSpec((ROW_BLK, 1), lambda i: (i, 0)),
        out_shape=jax.ShapeDtypeStruct((B, 1), jnp.float32),
        compiler_params=pltpu.CompilerParams(
            dimension_semantics=("parallel",),
        ),
    )(outputs, t2)

    loss = pl.pallas_call(
        _mean_kernel,
        in_specs=[pl.BlockSpec((B // 128, 128), lambda: (0, 0))],
        out_specs=pl.BlockSpec(memory_space=pltpu.SMEM),
        out_shape=jax.ShapeDtypeStruct((1, 1), jnp.float32),
    )(logp.reshape(B // 128, 128))
    return loss[0, 0]


# speedup vs baseline: 3.4325x; 1.0820x over previous
import jax
import jax.numpy as jnp
from jax.experimental import pallas as pl
from jax.experimental.pallas import tpu as pltpu

ROW_BLK = 128


def _ce_rows_kernel(x_ref, t_ref, out_ref):
    x = x_ref[...]
    t = t_ref[...]
    col = jax.lax.broadcasted_iota(jnp.int32, x.shape, 1)
    picked = jnp.sum(jnp.where(col == t, x, 0.0), axis=1, keepdims=True)
    m = jnp.max(x, axis=1, keepdims=True)
    s = jnp.sum(jnp.exp(x - m), axis=1, keepdims=True)
    out_ref[...] = picked - m - jnp.log(s)


def _mean_kernel(x_ref, out_ref):
    out_ref[0, 0] = -jnp.mean(x_ref[...])


def kernel(outputs, targets):
    B, V = outputs.shape
    t2 = targets.astype(jnp.int32).reshape(B, 1)
    logp = pl.pallas_call(
        _ce_rows_kernel,
        grid=(B // ROW_BLK,),
        in_specs=[
            pl.BlockSpec((ROW_BLK, V), lambda i: (i, 0)),
            pl.BlockSpec((ROW_BLK, 1), lambda i: (i, 0)),
        ],
        out_specs=pl.BlockSpec((ROW_BLK, 1), lambda i: (i, 0)),
        out_shape=jax.ShapeDtypeStruct((B, 1), jnp.float32),
        compiler_params=pltpu.CompilerParams(
            dimension_semantics=("parallel",),
            vmem_limit_bytes=56 * 1024 * 1024,
        ),
    )(outputs, t2)

    loss = pl.pallas_call(
        _mean_kernel,
        in_specs=[pl.BlockSpec((B // 128, 128), lambda: (0, 0))],
        out_specs=pl.BlockSpec(memory_space=pltpu.SMEM),
        out_shape=jax.ShapeDtypeStruct((1, 1), jnp.float32),
    )(logp.reshape(B // 128, 128))
    return loss[0, 0]


# speedup vs baseline: 3.6262x; 1.0564x over previous
import jax
import jax.numpy as jnp
from jax.experimental import pallas as pl
from jax.experimental.pallas import tpu as pltpu

ROW_BLK = 128


def _ce_rows_kernel(x1_ref, x2_ref, t_ref, out_ref):
    x1 = x1_ref[...]
    x2 = x2_ref[...]
    t = t_ref[...]
    half = x1.shape[1]
    col = jax.lax.broadcasted_iota(jnp.int32, x1.shape, 1)
    picked = jnp.sum(jnp.where(col == t, x1, 0.0), axis=1, keepdims=True)
    picked += jnp.sum(jnp.where(col + half == t, x2, 0.0), axis=1, keepdims=True)
    m = jnp.maximum(jnp.max(x1, axis=1, keepdims=True),
                    jnp.max(x2, axis=1, keepdims=True))
    s = (jnp.sum(jnp.exp(x1 - m), axis=1, keepdims=True)
         + jnp.sum(jnp.exp(x2 - m), axis=1, keepdims=True))
    out_ref[...] = picked - m - jnp.log(s)


def _mean_kernel(x_ref, out_ref):
    out_ref[0, 0] = -jnp.mean(x_ref[...])


def kernel(outputs, targets):
    B, V = outputs.shape
    half = V // 2
    t2 = targets.astype(jnp.int32).reshape(B, 1)
    logp = pl.pallas_call(
        _ce_rows_kernel,
        grid=(B // ROW_BLK,),
        in_specs=[
            pl.BlockSpec((ROW_BLK, half), lambda i: (i, 0)),
            pl.BlockSpec((ROW_BLK, half), lambda i: (i, 1)),
            pl.BlockSpec((ROW_BLK, 1), lambda i: (i, 0)),
        ],
        out_specs=pl.BlockSpec((ROW_BLK, 1), lambda i: (i, 0)),
        out_shape=jax.ShapeDtypeStruct((B, 1), jnp.float32),
        compiler_params=pltpu.CompilerParams(
            dimension_semantics=("parallel",),
            vmem_limit_bytes=56 * 1024 * 1024,
        ),
    )(outputs, outputs, t2)

    loss = pl.pallas_call(
        _mean_kernel,
        in_specs=[pl.BlockSpec((B // 128, 128), lambda: (0, 0))],
        out_specs=pl.BlockSpec(memory_space=pltpu.SMEM),
        out_shape=jax.ShapeDtypeStruct((1, 1), jnp.float32),
    )(logp.reshape(B // 128, 128))
    return loss[0, 0]


# speedup vs baseline: 3.6597x; 1.0092x over previous
import jax
import jax.numpy as jnp
from jax.experimental import pallas as pl
from jax.experimental.pallas import tpu as pltpu

ROW_BLK = 128
NSPLIT = 5


def _ce_rows_kernel(*refs):
    x_refs = refs[:NSPLIT]
    t_ref = refs[NSPLIT]
    out_ref = refs[NSPLIT + 1]
    t = t_ref[...]
    xs = [r[...] for r in x_refs]
    chunk = xs[0].shape[1]
    col = jax.lax.broadcasted_iota(jnp.int32, xs[0].shape, 1)
    m = xs[0].dtype.type(-jnp.inf)
    for x in xs:
        m = jnp.maximum(m, jnp.max(x, axis=1, keepdims=True))
    picked = jnp.zeros_like(m)
    s = jnp.zeros_like(m)
    for k, x in enumerate(xs):
        picked += jnp.sum(jnp.where(col + k * chunk == t, x, 0.0),
                          axis=1, keepdims=True)
        s += jnp.sum(jnp.exp(x - m), axis=1, keepdims=True)
    out_ref[...] = picked - m - jnp.log(s)


def _mean_kernel(x_ref, out_ref):
    out_ref[0, 0] = -jnp.mean(x_ref[...])


def kernel(outputs, targets):
    B, V = outputs.shape
    chunk = V // NSPLIT
    t2 = targets.astype(jnp.int32).reshape(B, 1)

    def vocab_spec(k):
        return pl.BlockSpec((ROW_BLK, chunk), lambda i, k=k: (i, k))

    logp = pl.pallas_call(
        _ce_rows_kernel,
        grid=(B // ROW_BLK,),
        in_specs=[vocab_spec(k) for k in range(NSPLIT)]
        + [pl.BlockSpec((ROW_BLK, 1), lambda i: (i, 0))],
        out_specs=pl.BlockSpec((ROW_BLK, 1), lambda i: (i, 0)),
        out_shape=jax.ShapeDtypeStruct((B, 1), jnp.float32),
        compiler_params=pltpu.CompilerParams(
            dimension_semantics=("parallel",),
            vmem_limit_bytes=56 * 1024 * 1024,
        ),
    )(*([outputs] * NSPLIT), t2)

    loss = pl.pallas_call(
        _mean_kernel,
        in_specs=[pl.BlockSpec((B // 128, 128), lambda: (0, 0))],
        out_specs=pl.BlockSpec(memory_space=pltpu.SMEM),
        out_shape=jax.ShapeDtypeStruct((1, 1), jnp.float32),
    )(logp.reshape(B // 128, 128))
    return loss[0, 0]


# speedup vs baseline: 3.6744x; 1.0040x over previous
import functools

import jax
import jax.numpy as jnp
from jax.experimental import pallas as pl
from jax.experimental.pallas import tpu as pltpu

ROW_BLK = 128
NSPLIT = 5


def _ce_rows_kernel(*refs):
    x_refs = refs[:NSPLIT]
    t_ref = refs[NSPLIT]
    out_ref = refs[NSPLIT + 1]
    t = t_ref[...]
    xs = [r[...] for r in x_refs]
    chunk = xs[0].shape[1]
    col = jax.lax.broadcasted_iota(jnp.int32, xs[0].shape, 1)
    m = xs[0].dtype.type(-jnp.inf)
    for x in xs:
        m = jnp.maximum(m, jnp.max(x, axis=1, keepdims=True))
    picked = jnp.zeros_like(m)
    s = jnp.zeros_like(m)
    for k, x in enumerate(xs):
        picked += jnp.sum(jnp.where(col + k * chunk == t, x, 0.0),
                          axis=1, keepdims=True)
        s += jnp.sum(jnp.exp(x - m), axis=1, keepdims=True)
    out_ref[...] = jnp.sum(picked - m - jnp.log(s)).reshape(1, 1, 1)


def _mean_kernel(x_ref, out_ref, *, n_rows):
    out_ref[0, 0] = -jnp.sum(x_ref[...]) / n_rows


def kernel(outputs, targets):
    B, V = outputs.shape
    chunk = V // NSPLIT
    t2 = targets.astype(jnp.int32).reshape(B, 1)

    def vocab_spec(k):
        return pl.BlockSpec((ROW_BLK, chunk), lambda i, k=k: (i, k))

    nblk = B // ROW_BLK
    partial = pl.pallas_call(
        _ce_rows_kernel,
        grid=(nblk,),
        in_specs=[vocab_spec(k) for k in range(NSPLIT)]
        + [pl.BlockSpec((ROW_BLK, 1), lambda i: (i, 0))],
        out_specs=pl.BlockSpec((1, 1, 1), lambda i: (i, 0, 0)),
        out_shape=jax.ShapeDtypeStruct((nblk, 1, 1), jnp.float32),
        compiler_params=pltpu.CompilerParams(
            dimension_semantics=("parallel",),
            vmem_limit_bytes=56 * 1024 * 1024,
        ),
    )(*([outputs] * NSPLIT), t2)

    loss = pl.pallas_call(
        functools.partial(_mean_kernel, n_rows=B),
        in_specs=[pl.BlockSpec((nblk, 1, 1), lambda: (0, 0, 0))],
        out_specs=pl.BlockSpec(memory_space=pltpu.SMEM),
        out_shape=jax.ShapeDtypeStruct((1, 1), jnp.float32),
    )(partial)
    return loss[0, 0]
